# baseline (speedup 1.0000x reference)
import functools

import jax
import jax.numpy as jnp
from jax import lax
from jax.experimental import pallas as pl
from jax.experimental.pallas import tpu as pltpu
from jax.experimental.pallas import tpu_sc as plsc

_NUM_GRAPHS = 128
_SC_CORES = 2
_SC_SUBCORES = 16
_SC_WORKERS = _SC_CORES * _SC_SUBCORES
_K = 80
_J = 5
_CHUNK = _K * _J
_SK = 40
_SJ = 5
_SCHUNK = _SK * _SJ


def _sc_gather(table, idx, e_off, e_len):
    e = e_len
    per_w = e // _SC_WORKERS
    mesh = plsc.VectorSubcoreMesh(core_axis_name="c", subcore_axis_name="s")

    @functools.partial(
        pl.kernel,
        mesh=mesh,
        out_type=jax.ShapeDtypeStruct((e, 128), jnp.float32),
        scratch_types=[
            pltpu.VMEM((_CHUNK,), jnp.int32),
            pltpu.VMEM((_CHUNK, 128), jnp.float32),
            pltpu.SemaphoreType.DMA,
        ],
    )
    def k(table_hbm, idx_hbm, out_hbm, idx_v, rows_v, sem):
        wid = lax.axis_index("s") * _SC_CORES + lax.axis_index("c")
        base = wid * per_w

        @pl.loop(0, per_w // _CHUNK)
        def _(t):
            pltpu.sync_copy(
                idx_hbm.at[pl.ds(e_off + base + t * _CHUNK, _CHUNK)],
                idx_v)
            handles = [
                pltpu.async_copy(table_hbm.at[idx_v.at[pl.ds(j * _K, _K)]],
                                 rows_v.at[pl.ds(j * _K, _K)], sem)
                for j in range(_J)
            ]
            for h in handles:
                h.wait()
            pltpu.sync_copy(rows_v,
                            out_hbm.at[pl.ds(base + t * _CHUNK, _CHUNK)])

    return k(table, idx)


def _sc_scatter_add(msg, dst, e_off, zeros):
    e, _ = msg.shape
    n_pad = zeros.shape[0]
    per_w = e // _SC_WORKERS
    rows_per_sub = n_pad // _SC_SUBCORES
    mesh = plsc.VectorSubcoreMesh(core_axis_name="c", subcore_axis_name="s")

    @functools.partial(
        pl.kernel,
        mesh=mesh,
        out_type=jax.ShapeDtypeStruct((2 * n_pad, 128), jnp.float32),
        scratch_types=[
            pltpu.VMEM((_SJ, _SK), jnp.int32),
            pltpu.VMEM((_SCHUNK, 128), jnp.float32),
            pltpu.VMEM_SHARED((n_pad, 128), jnp.float32),
            pltpu.SemaphoreType.DMA,
            pltpu.SemaphoreType.DMA,
        ],
    )
    def k(msg_hbm, dst_hbm, zero_hbm, out_hbm, idx_v, rows_v, acc_sh, sem,
          isem):
        cid = lax.axis_index("c")
        sid = lax.axis_index("s")
        wid = sid * _SC_CORES + cid
        base = wid * per_w
        my_rows = sid * rows_per_sub

        pltpu.sync_copy(zero_hbm.at[pl.ds(my_rows, rows_per_sub)],
                        acc_sh.at[pl.ds(my_rows, rows_per_sub)])
        plsc.subcore_barrier()

        @pl.loop(0, per_w // _SCHUNK)
        def _(t):
            ih = [
                pltpu.async_copy(
                    dst_hbm.at[
                        pl.ds(e_off + base + t * _SCHUNK + j * _SK, _SK)],
                    idx_v.at[j], isem)
                for j in range(_SJ)
            ]
            pltpu.sync_copy(msg_hbm.at[pl.ds(base + t * _SCHUNK, _SCHUNK)],
                            rows_v)
            for h in ih:
                h.wait()
            for j in range(_SJ):
                pltpu.sync_copy(rows_v.at[pl.ds(j * _SK, _SK)],
                                acc_sh.at[idx_v.at[j]], add=True)

        plsc.subcore_barrier()
        pltpu.sync_copy(
            acc_sh.at[pl.ds(my_rows, rows_per_sub)],
            out_hbm.at[pl.ds(cid * n_pad + my_rows, rows_per_sub)])

    return k(msg, dst, zeros)


def _edge_body(in_c, out_c, blk, ea_ref, xj_ref, w1_ref, b1_ref, w2_ref,
               b2_ref, rep_ref, sum_ref, out_ref):
    h = jnp.maximum(
        jnp.dot(ea_ref[...], w1_ref[...],
                preferred_element_type=jnp.float32) + b1_ref[...], 0.0)
    g = jnp.dot(h, w2_ref[...],
                preferred_element_type=jnp.float32) + b2_ref[...]
    xjr = jnp.dot(xj_ref[...].astype(jnp.bfloat16), rep_ref[...],
                  preferred_element_type=jnp.float32)
    msg = jnp.dot((xjr * g).astype(jnp.bfloat16), sum_ref[...],
                  preferred_element_type=jnp.float32)
    out_ref[...] = jnp.concatenate(
        [msg, jnp.zeros((blk, 128 - out_c), jnp.float32)], axis=1)


def _edge_messages(ea, xj, w1, b1, w2, b2, in_c, out_c, blk, blk_off):
    e = xj.shape[0]
    hid = w1.shape[1]
    ic_oc = in_c * out_c
    rep = (jnp.arange(ic_oc)[None, :] // out_c
           == jnp.arange(128)[:, None]).astype(jnp.bfloat16)
    summ = (jnp.arange(ic_oc)[:, None] % out_c
            == jnp.arange(out_c)[None, :]).astype(jnp.bfloat16)
    kfn = functools.partial(_edge_body, in_c, out_c, blk)
    return pl.pallas_call(
        kfn,
        grid=(e // blk,),
        in_specs=[
            pl.BlockSpec((blk, ea.shape[1]), lambda i: (i + blk_off, 0)),
            pl.BlockSpec((blk, 128), lambda i: (i, 0)),
            pl.BlockSpec(w1.shape, lambda i: (0, 0)),
            pl.BlockSpec((1, hid), lambda i: (0, 0)),
            pl.BlockSpec(w2.shape, lambda i: (0, 0)),
            pl.BlockSpec((1, ic_oc), lambda i: (0, 0)),
            pl.BlockSpec((128, ic_oc), lambda i: (0, 0)),
            pl.BlockSpec((ic_oc, out_c), lambda i: (0, 0)),
        ],
        out_specs=pl.BlockSpec((blk, 128), lambda i: (i, 0)),
        out_shape=jax.ShapeDtypeStruct((e, 128), jnp.float32),
    )(ea, xj, w1, b1[None, :], w2, b2[None, :], rep, summ)


def _node1_body(nb, pa_ref, pb_ref, x_ref, r_ref, b_ref, o_ref):
    agg = (pa_ref[0, :, 0:32] + pa_ref[1, :, 0:32]
           + pb_ref[0, :, 0:32] + pb_ref[1, :, 0:32])
    h = jnp.maximum(
        agg + jnp.dot(x_ref[...], r_ref[...],
                      preferred_element_type=jnp.float32) + b_ref[...], 0.0)
    o_ref[...] = jnp.concatenate(
        [h, jnp.zeros((nb, 96), jnp.float32)], axis=1)


def _node1(parts_a, parts_b, x, root, bias, nb=1000):
    n = x.shape[0]
    n_pad = parts_a.shape[0] // 2
    pa = parts_a.reshape(2, n_pad, 128)
    pb = parts_b.reshape(2, n_pad, 128)
    return pl.pallas_call(
        functools.partial(_node1_body, nb),
        grid=(n // nb,),
        in_specs=[
            pl.BlockSpec((2, nb, 128), lambda i: (0, i, 0)),
            pl.BlockSpec((2, nb, 128), lambda i: (0, i, 0)),
            pl.BlockSpec((nb, x.shape[1]), lambda i: (i, 0)),
            pl.BlockSpec(root.shape, lambda i: (0, 0)),
            pl.BlockSpec((1, 32), lambda i: (0, 0)),
        ],
        out_specs=pl.BlockSpec((nb, 128), lambda i: (i, 0)),
        out_shape=jax.ShapeDtypeStruct((n, 128), jnp.float32),
    )(pa, pb, x, root, bias[None, :])


def _split_body(ei_ref, s_ref, d_ref):
    s_ref[...] = ei_ref[0]
    d_ref[...] = ei_ref[1]


def _split_edge_index(edge_index, blk=128000):
    e = edge_index.shape[1]
    out = jax.ShapeDtypeStruct((e,), jnp.int32)
    return pl.pallas_call(
        _split_body,
        grid=(e // blk,),
        in_specs=[pl.BlockSpec((2, blk), lambda i: (0, i))],
        out_specs=[pl.BlockSpec((blk,), lambda i: (i,)),
                   pl.BlockSpec((blk,), lambda i: (i,))],
        out_shape=[out, out],
    )(edge_index)


def _pool_body(nb, pa_ref, pb_ref, h_ref, r_ref, b_ref, batch_ref, o_ref):
    i = pl.program_id(0)
    agg = (pa_ref[0, :, 0:16] + pa_ref[1, :, 0:16]
           + pb_ref[0, :, 0:16] + pb_ref[1, :, 0:16])
    h2 = jnp.maximum(
        agg + jnp.dot(h_ref[:, 0:32], r_ref[...],
                      preferred_element_type=jnp.float32) + b_ref[...], 0.0)
    seg = lax.broadcasted_iota(jnp.int32, (_NUM_GRAPHS, nb), 0)
    onehot = jnp.where(seg == batch_ref[0], 1.0, 0.0)
    g = jnp.dot(onehot, h2, preferred_element_type=jnp.float32)

    @pl.when(i == 0)
    def _():
        o_ref[...] = jnp.zeros_like(o_ref)

    o_ref[...] += g


def _pool(parts2_a, parts2_b, h, root2, bias2, batch, nb=1000):
    n = h.shape[0]
    n_pad = parts2_a.shape[0] // 2
    pa = parts2_a.reshape(2, n_pad, 128)
    pb = parts2_b.reshape(2, n_pad, 128)
    return pl.pallas_call(
        functools.partial(_pool_body, nb),
        grid=(n // nb,),
        in_specs=[
            pl.BlockSpec((2, nb, 128), lambda i: (0, i, 0)),
            pl.BlockSpec((2, nb, 128), lambda i: (0, i, 0)),
            pl.BlockSpec((nb, 128), lambda i: (i, 0)),
            pl.BlockSpec(root2.shape, lambda i: (0, 0)),
            pl.BlockSpec((1, 16), lambda i: (0, 0)),
            pl.BlockSpec((1, 1, nb), lambda i: (i, 0, 0)),
        ],
        out_specs=pl.BlockSpec((_NUM_GRAPHS, 16), lambda i: (0, 0)),
        out_shape=jax.ShapeDtypeStruct((_NUM_GRAPHS, 16), jnp.float32),
    )(pa, pb, h, root2, bias2[None, :], batch.reshape(n // nb, 1, nb))


def _mlp_body(g_ref, fw_ref, fb_ref, ow_ref, ob_ref, o_ref):
    g = jnp.maximum(
        jnp.dot(g_ref[...], fw_ref[...], preferred_element_type=jnp.float32)
        + fb_ref[...], 0.0)
    o_ref[...] = jnp.dot(g, ow_ref[...],
                         preferred_element_type=jnp.float32) + ob_ref[...]


def _mlp(g, fc1_w, fc1_b, out_w, out_b):
    return pl.pallas_call(
        _mlp_body,
        out_shape=jax.ShapeDtypeStruct((_NUM_GRAPHS, 1), jnp.float32),
    )(g, fc1_w, fc1_b[None, :], out_w, out_b[None, :])


def kernel(x, edge_index, edge_attr, batch, c1_W1, c1_b1, c1_W2, c1_b2,
           c1_root, c1_bias, c2_W1, c2_b1, c2_W2, c2_b2, c2_root, c2_bias,
           fc1_W, fc1_b, out_W, out_b):
    n = x.shape[0]
    e = edge_index.shape[1]
    eh = e // 2
    blk = 2000
    n_pad = -(-n // (8 * _SC_SUBCORES)) * (8 * _SC_SUBCORES)
    src, dst = _split_edge_index(edge_index)

    x128 = jnp.pad(x, ((0, 0), (0, 128 - x.shape[1])))
    zeros = jnp.zeros((n_pad, 128), jnp.float32)

    p1 = []
    for s in range(2):
        xj = _sc_gather(x128, src, s * eh, eh)
        msg1 = _edge_messages(edge_attr, xj, c1_W1, c1_b1, c1_W2, c1_b2,
                              in_c=11, out_c=32, blk=blk,
                              blk_off=s * eh // blk)
        p1.append(_sc_scatter_add(msg1, dst, s * eh, zeros))
    h = _node1(p1[0], p1[1], x, c1_root, c1_bias)

    p2 = []
    for s in range(2):
        hj = _sc_gather(h, src, s * eh, eh)
        msg2 = _edge_messages(edge_attr, hj, c2_W1, c2_b1, c2_W2, c2_b2,
                              in_c=32, out_c=16, blk=blk,
                              blk_off=s * eh // blk)
        p2.append(_sc_scatter_add(msg2, dst, s * eh, zeros))
    g = _pool(p2[0], p2[1], h, c2_root, c2_bias, batch)
    return _mlp(g, fc1_W, fc1_b, out_W, out_b)


# speedup vs baseline: 1.1342x; 1.1342x over previous
import functools

import jax
import jax.numpy as jnp
from jax import lax
from jax.experimental import pallas as pl
from jax.experimental.pallas import tpu as pltpu
from jax.experimental.pallas import tpu_sc as plsc

_NUM_GRAPHS = 128
_SC_CORES = 2
_SC_SUBCORES = 16
_SC_WORKERS = _SC_CORES * _SC_SUBCORES
_NSTREAM = 4
_K = 40
_J = 25
_CHUNK = _K * _J
_SK = 40
_SJ = 5
_SCHUNK = _SK * _SJ


def _sc_gather(table, idx, e_off, e_len):
    e = e_len
    per_w = e // _SC_WORKERS
    mesh = plsc.VectorSubcoreMesh(core_axis_name="c", subcore_axis_name="s")

    @functools.partial(
        pl.kernel,
        mesh=mesh,
        out_type=jax.ShapeDtypeStruct((e, 128), jnp.float32),
        scratch_types=[
            pltpu.VMEM((_CHUNK,), jnp.int32),
            pltpu.VMEM((_CHUNK, 128), jnp.float32),
            pltpu.SemaphoreType.DMA,
        ],
    )
    def k(table_hbm, idx_hbm, out_hbm, idx_v, rows_v, sem):
        wid = lax.axis_index("s") * _SC_CORES + lax.axis_index("c")
        base = wid * per_w

        @pl.loop(0, per_w // _CHUNK)
        def _(t):
            pltpu.sync_copy(
                idx_hbm.at[pl.ds(e_off + base + t * _CHUNK, _CHUNK)],
                idx_v)
            handles = [
                pltpu.async_copy(table_hbm.at[idx_v.at[pl.ds(j * _K, _K)]],
                                 rows_v.at[pl.ds(j * _K, _K)], sem)
                for j in range(_J)
            ]
            for h in handles:
                h.wait()
            pltpu.sync_copy(rows_v,
                            out_hbm.at[pl.ds(base + t * _CHUNK, _CHUNK)])

    return k(table, idx)


def _sc_scatter_add(msg, dst, e_off, zeros):
    e, _ = msg.shape
    n_pad = zeros.shape[0]
    per_w = e // _SC_WORKERS
    rows_per_sub = n_pad // _SC_SUBCORES
    mesh = plsc.VectorSubcoreMesh(core_axis_name="c", subcore_axis_name="s")

    @functools.partial(
        pl.kernel,
        mesh=mesh,
        out_type=jax.ShapeDtypeStruct((2 * n_pad, 128), jnp.float32),
        scratch_types=[
            pltpu.VMEM((_SJ, _SK), jnp.int32),
            pltpu.VMEM((_SCHUNK, 128), jnp.float32),
            pltpu.VMEM_SHARED((n_pad, 128), jnp.float32),
            pltpu.SemaphoreType.DMA,
            pltpu.SemaphoreType.DMA,
        ],
    )
    def k(msg_hbm, dst_hbm, zero_hbm, out_hbm, idx_v, rows_v, acc_sh, sem,
          isem):
        cid = lax.axis_index("c")
        sid = lax.axis_index("s")
        wid = sid * _SC_CORES + cid
        base = wid * per_w
        my_rows = sid * rows_per_sub

        pltpu.sync_copy(zero_hbm.at[pl.ds(my_rows, rows_per_sub)],
                        acc_sh.at[pl.ds(my_rows, rows_per_sub)])
        plsc.subcore_barrier()

        @pl.loop(0, per_w // _SCHUNK)
        def _(t):
            ih = [
                pltpu.async_copy(
                    dst_hbm.at[
                        pl.ds(e_off + base + t * _SCHUNK + j * _SK, _SK)],
                    idx_v.at[j], isem)
                for j in range(_SJ)
            ]
            pltpu.sync_copy(msg_hbm.at[pl.ds(base + t * _SCHUNK, _SCHUNK)],
                            rows_v)
            for h in ih:
                h.wait()
            for j in range(_SJ):
                pltpu.sync_copy(rows_v.at[pl.ds(j * _SK, _SK)],
                                acc_sh.at[idx_v.at[j]], add=True)

        plsc.subcore_barrier()
        pltpu.sync_copy(
            acc_sh.at[pl.ds(my_rows, rows_per_sub)],
            out_hbm.at[pl.ds(cid * n_pad + my_rows, rows_per_sub)])

    return k(msg, dst, zeros)


def _edge_body(in_c, out_c, blk, ea_ref, xj_ref, w1_ref, b1_ref, w2_ref,
               b2_ref, rep_ref, sum_ref, out_ref):
    h = jnp.maximum(
        jnp.dot(ea_ref[...], w1_ref[...],
                preferred_element_type=jnp.float32) + b1_ref[...], 0.0)
    g = jnp.dot(h, w2_ref[...],
                preferred_element_type=jnp.float32) + b2_ref[...]
    xjr = jnp.dot(xj_ref[...], rep_ref[...],
                  preferred_element_type=jnp.float32)
    msg = jnp.dot(xjr * g, sum_ref[...], preferred_element_type=jnp.float32)
    out_ref[...] = jnp.concatenate(
        [msg, jnp.zeros((blk, 128 - out_c), jnp.float32)], axis=1)


def _edge_messages(ea, xj, w1, b1, w2, b2, in_c, out_c, blk, blk_off):
    e = xj.shape[0]
    hid = w1.shape[1]
    ic_oc = in_c * out_c
    rep = (jnp.arange(ic_oc)[None, :] // out_c
           == jnp.arange(128)[:, None]).astype(jnp.float32)
    summ = (jnp.arange(ic_oc)[:, None] % out_c
            == jnp.arange(out_c)[None, :]).astype(jnp.float32)
    kfn = functools.partial(_edge_body, in_c, out_c, blk)
    return pl.pallas_call(
        kfn,
        grid=(e // blk,),
        in_specs=[
            pl.BlockSpec((blk, ea.shape[1]), lambda i: (i + blk_off, 0)),
            pl.BlockSpec((blk, 128), lambda i: (i, 0)),
            pl.BlockSpec(w1.shape, lambda i: (0, 0)),
            pl.BlockSpec((1, hid), lambda i: (0, 0)),
            pl.BlockSpec(w2.shape, lambda i: (0, 0)),
            pl.BlockSpec((1, ic_oc), lambda i: (0, 0)),
            pl.BlockSpec((128, ic_oc), lambda i: (0, 0)),
            pl.BlockSpec((ic_oc, out_c), lambda i: (0, 0)),
        ],
        out_specs=pl.BlockSpec((blk, 128), lambda i: (i, 0)),
        out_shape=jax.ShapeDtypeStruct((e, 128), jnp.float32),
    )(ea, xj, w1, b1[None, :], w2, b2[None, :], rep, summ)


def _node1_body(nb, ns, *refs):
    p_refs = refs[:ns]
    x_ref, r_ref, b_ref, o_ref = refs[ns:]
    agg = sum(p[0, :, 0:32] + p[1, :, 0:32] for p in p_refs)
    h = jnp.maximum(
        agg + jnp.dot(x_ref[...], r_ref[...],
                      preferred_element_type=jnp.float32) + b_ref[...], 0.0)
    o_ref[...] = jnp.concatenate(
        [h, jnp.zeros((nb, 96), jnp.float32)], axis=1)


def _node1(parts, x, root, bias, nb=1000):
    n = x.shape[0]
    n_pad = parts[0].shape[0] // 2
    ps = [p.reshape(2, n_pad, 128) for p in parts]
    return pl.pallas_call(
        functools.partial(_node1_body, nb, len(ps)),
        grid=(n // nb,),
        in_specs=(
            [pl.BlockSpec((2, nb, 128), lambda i: (0, i, 0))] * len(ps) + [
            pl.BlockSpec((nb, x.shape[1]), lambda i: (i, 0)),
            pl.BlockSpec(root.shape, lambda i: (0, 0)),
            pl.BlockSpec((1, 32), lambda i: (0, 0)),
        ]),
        out_specs=pl.BlockSpec((nb, 128), lambda i: (i, 0)),
        out_shape=jax.ShapeDtypeStruct((n, 128), jnp.float32),
    )(*ps, x, root, bias[None, :])


def _split_body(ei_ref, s_ref, d_ref):
    s_ref[...] = ei_ref[0]
    d_ref[...] = ei_ref[1]


def _split_edge_index(edge_index, blk=128000):
    e = edge_index.shape[1]
    out = jax.ShapeDtypeStruct((e,), jnp.int32)
    return pl.pallas_call(
        _split_body,
        grid=(e // blk,),
        in_specs=[pl.BlockSpec((2, blk), lambda i: (0, i))],
        out_specs=[pl.BlockSpec((blk,), lambda i: (i,)),
                   pl.BlockSpec((blk,), lambda i: (i,))],
        out_shape=[out, out],
    )(edge_index)


def _pool_body(nb, ns, *refs):
    p_refs = refs[:ns]
    h_ref, r_ref, b_ref, batch_ref, o_ref = refs[ns:]
    i = pl.program_id(0)
    agg = sum(p[0, :, 0:16] + p[1, :, 0:16] for p in p_refs)
    h2 = jnp.maximum(
        agg + jnp.dot(h_ref[:, 0:32], r_ref[...],
                      preferred_element_type=jnp.float32) + b_ref[...], 0.0)
    seg = lax.broadcasted_iota(jnp.int32, (_NUM_GRAPHS, nb), 0)
    onehot = jnp.where(seg == batch_ref[0], 1.0, 0.0)
    g = jnp.dot(onehot, h2, preferred_element_type=jnp.float32)

    @pl.when(i == 0)
    def _():
        o_ref[...] = jnp.zeros_like(o_ref)

    o_ref[...] += g


def _pool(parts2, h, root2, bias2, batch, nb=1000):
    n = h.shape[0]
    n_pad = parts2[0].shape[0] // 2
    ps = [p.reshape(2, n_pad, 128) for p in parts2]
    return pl.pallas_call(
        functools.partial(_pool_body, nb, len(ps)),
        grid=(n // nb,),
        in_specs=(
            [pl.BlockSpec((2, nb, 128), lambda i: (0, i, 0))] * len(ps) + [
            pl.BlockSpec((nb, 128), lambda i: (i, 0)),
            pl.BlockSpec(root2.shape, lambda i: (0, 0)),
            pl.BlockSpec((1, 16), lambda i: (0, 0)),
            pl.BlockSpec((1, 1, nb), lambda i: (i, 0, 0)),
        ]),
        out_specs=pl.BlockSpec((_NUM_GRAPHS, 16), lambda i: (0, 0)),
        out_shape=jax.ShapeDtypeStruct((_NUM_GRAPHS, 16), jnp.float32),
    )(*ps, h, root2, bias2[None, :], batch.reshape(n // nb, 1, nb))


def _mlp_body(g_ref, fw_ref, fb_ref, ow_ref, ob_ref, o_ref):
    g = jnp.maximum(
        jnp.dot(g_ref[...], fw_ref[...], preferred_element_type=jnp.float32)
        + fb_ref[...], 0.0)
    o_ref[...] = jnp.dot(g, ow_ref[...],
                         preferred_element_type=jnp.float32) + ob_ref[...]


def _mlp(g, fc1_w, fc1_b, out_w, out_b):
    return pl.pallas_call(
        _mlp_body,
        out_shape=jax.ShapeDtypeStruct((_NUM_GRAPHS, 1), jnp.float32),
    )(g, fc1_w, fc1_b[None, :], out_w, out_b[None, :])


def kernel(x, edge_index, edge_attr, batch, c1_W1, c1_b1, c1_W2, c1_b2,
           c1_root, c1_bias, c2_W1, c2_b1, c2_W2, c2_b2, c2_root, c2_bias,
           fc1_W, fc1_b, out_W, out_b):
    n = x.shape[0]
    e = edge_index.shape[1]
    eh = e // _NSTREAM
    blk = 2000
    n_pad = -(-n // (8 * _SC_SUBCORES)) * (8 * _SC_SUBCORES)
    src, dst = _split_edge_index(edge_index)

    x128 = jnp.pad(x, ((0, 0), (0, 128 - x.shape[1])))
    zeros = jnp.zeros((n_pad, 128), jnp.float32)

    p1 = []
    for s in range(_NSTREAM):
        xj = _sc_gather(x128, src, s * eh, eh)
        msg1 = _edge_messages(edge_attr, xj, c1_W1, c1_b1, c1_W2, c1_b2,
                              in_c=11, out_c=32, blk=blk,
                              blk_off=s * eh // blk)
        p1.append(_sc_scatter_add(msg1, dst, s * eh, zeros))
    h = _node1(p1, x, c1_root, c1_bias)

    p2 = []
    for s in range(_NSTREAM):
        hj = _sc_gather(h, src, s * eh, eh)
        msg2 = _edge_messages(edge_attr, hj, c2_W1, c2_b1, c2_W2, c2_b2,
                              in_c=32, out_c=16, blk=blk,
                              blk_off=s * eh // blk)
        p2.append(_sc_scatter_add(msg2, dst, s * eh, zeros))
    g = _pool(p2, h, c2_root, c2_bias, batch)
    return _mlp(g, fc1_W, fc1_b, out_W, out_b)


# speedup vs baseline: 1.2107x; 1.0674x over previous
import functools

import jax
import jax.numpy as jnp
from jax import lax
from jax.experimental import pallas as pl
from jax.experimental.pallas import tpu as pltpu
from jax.experimental.pallas import tpu_sc as plsc

_NUM_GRAPHS = 128
_SC_CORES = 2
_SC_SUBCORES = 16
_SC_WORKERS = _SC_CORES * _SC_SUBCORES
_NSTREAM = 4
_K = 40
_J = 5
_CHUNK = _K * _J
_SK = 40
_SJ = 5
_SCHUNK = _SK * _SJ


def _sc_gather(table, idx, e_off, e_len):
    e = e_len
    n_rows = table.shape[0]
    per_w = e // _SC_WORKERS
    stage = 640
    mesh = plsc.VectorSubcoreMesh(core_axis_name="c", subcore_axis_name="s")

    @functools.partial(
        pl.kernel,
        mesh=mesh,
        out_type=jax.ShapeDtypeStruct((e, 128), jnp.float32),
        scratch_types=[
            pltpu.VMEM((_CHUNK,), jnp.int32),
            pltpu.VMEM((_CHUNK, 128), jnp.float32),
            pltpu.VMEM_SHARED((n_rows, 128), jnp.float32),
            pltpu.SemaphoreType.DMA,
        ],
    )
    def k(table_hbm, idx_hbm, out_hbm, idx_v, rows_v, tab_sh, sem):
        cid = lax.axis_index("c")
        sid = lax.axis_index("s")
        wid = sid * _SC_CORES + cid
        base = wid * per_w

        full = n_rows // stage
        tail = n_rows - full * stage

        @pl.when(sid < full)
        def _():
            pltpu.sync_copy(table_hbm.at[pl.ds(sid * stage, stage)],
                            tab_sh.at[pl.ds(sid * stage, stage)])

        if tail:
            @pl.when(sid == full)
            def _():
                pltpu.sync_copy(table_hbm.at[pl.ds(full * stage, tail)],
                                tab_sh.at[pl.ds(full * stage, tail)])
        plsc.subcore_barrier()

        @pl.loop(0, per_w // _CHUNK)
        def _(t):
            pltpu.sync_copy(
                idx_hbm.at[pl.ds(e_off + base + t * _CHUNK, _CHUNK)],
                idx_v)
            handles = [
                pltpu.async_copy(tab_sh.at[idx_v.at[pl.ds(j * _K, _K)]],
                                 rows_v.at[pl.ds(j * _K, _K)], sem)
                for j in range(_J)
            ]
            for h in handles:
                h.wait()
            pltpu.sync_copy(rows_v,
                            out_hbm.at[pl.ds(base + t * _CHUNK, _CHUNK)])

    return k(table, idx)


def _sc_scatter_add(msg, dst, e_off, zeros):
    e, _ = msg.shape
    n_pad = zeros.shape[0]
    per_w = e // _SC_WORKERS
    rows_per_sub = n_pad // _SC_SUBCORES
    mesh = plsc.VectorSubcoreMesh(core_axis_name="c", subcore_axis_name="s")

    @functools.partial(
        pl.kernel,
        mesh=mesh,
        out_type=jax.ShapeDtypeStruct((2 * n_pad, 128), jnp.float32),
        scratch_types=[
            pltpu.VMEM((_SJ, _SK), jnp.int32),
            pltpu.VMEM((_SCHUNK, 128), jnp.float32),
            pltpu.VMEM_SHARED((n_pad, 128), jnp.float32),
            pltpu.SemaphoreType.DMA,
            pltpu.SemaphoreType.DMA,
        ],
    )
    def k(msg_hbm, dst_hbm, zero_hbm, out_hbm, idx_v, rows_v, acc_sh, sem,
          isem):
        cid = lax.axis_index("c")
        sid = lax.axis_index("s")
        wid = sid * _SC_CORES + cid
        base = wid * per_w
        my_rows = sid * rows_per_sub

        pltpu.sync_copy(zero_hbm.at[pl.ds(my_rows, rows_per_sub)],
                        acc_sh.at[pl.ds(my_rows, rows_per_sub)])
        plsc.subcore_barrier()

        @pl.loop(0, per_w // _SCHUNK)
        def _(t):
            ih = [
                pltpu.async_copy(
                    dst_hbm.at[
                        pl.ds(e_off + base + t * _SCHUNK + j * _SK, _SK)],
                    idx_v.at[j], isem)
                for j in range(_SJ)
            ]
            pltpu.sync_copy(msg_hbm.at[pl.ds(base + t * _SCHUNK, _SCHUNK)],
                            rows_v)
            for h in ih:
                h.wait()
            for j in range(_SJ):
                pltpu.sync_copy(rows_v.at[pl.ds(j * _SK, _SK)],
                                acc_sh.at[idx_v.at[j]], add=True)

        plsc.subcore_barrier()
        pltpu.sync_copy(
            acc_sh.at[pl.ds(my_rows, rows_per_sub)],
            out_hbm.at[pl.ds(cid * n_pad + my_rows, rows_per_sub)])

    return k(msg, dst, zeros)


def _edge_body(in_c, out_c, blk, ea_ref, xj_ref, w1_ref, b1_ref, w2_ref,
               b2_ref, rep_ref, sum_ref, out_ref):
    h = jnp.maximum(
        jnp.dot(ea_ref[...], w1_ref[...],
                preferred_element_type=jnp.float32) + b1_ref[...], 0.0)
    g = jnp.dot(h, w2_ref[...],
                preferred_element_type=jnp.float32) + b2_ref[...]
    xjr = jnp.dot(xj_ref[...], rep_ref[...],
                  preferred_element_type=jnp.float32)
    msg = jnp.dot(xjr * g, sum_ref[...], preferred_element_type=jnp.float32)
    out_ref[...] = jnp.concatenate(
        [msg, jnp.zeros((blk, 128 - out_c), jnp.float32)], axis=1)


def _edge_messages(ea, xj, w1, b1, w2, b2, in_c, out_c, blk, blk_off):
    e = xj.shape[0]
    hid = w1.shape[1]
    ic_oc = in_c * out_c
    rep = (jnp.arange(ic_oc)[None, :] // out_c
           == jnp.arange(128)[:, None]).astype(jnp.float32)
    summ = (jnp.arange(ic_oc)[:, None] % out_c
            == jnp.arange(out_c)[None, :]).astype(jnp.float32)
    kfn = functools.partial(_edge_body, in_c, out_c, blk)
    return pl.pallas_call(
        kfn,
        grid=(e // blk,),
        in_specs=[
            pl.BlockSpec((blk, ea.shape[1]), lambda i: (i + blk_off, 0)),
            pl.BlockSpec((blk, 128), lambda i: (i, 0)),
            pl.BlockSpec(w1.shape, lambda i: (0, 0)),
            pl.BlockSpec((1, hid), lambda i: (0, 0)),
            pl.BlockSpec(w2.shape, lambda i: (0, 0)),
            pl.BlockSpec((1, ic_oc), lambda i: (0, 0)),
            pl.BlockSpec((128, ic_oc), lambda i: (0, 0)),
            pl.BlockSpec((ic_oc, out_c), lambda i: (0, 0)),
        ],
        out_specs=pl.BlockSpec((blk, 128), lambda i: (i, 0)),
        out_shape=jax.ShapeDtypeStruct((e, 128), jnp.float32),
    )(ea, xj, w1, b1[None, :], w2, b2[None, :], rep, summ)


def _node1_body(nb, ns, *refs):
    p_refs = refs[:ns]
    x_ref, r_ref, b_ref, o_ref = refs[ns:]
    agg = sum(p[0, :, 0:32] + p[1, :, 0:32] for p in p_refs)
    h = jnp.maximum(
        agg + jnp.dot(x_ref[...], r_ref[...],
                      preferred_element_type=jnp.float32) + b_ref[...], 0.0)
    o_ref[...] = jnp.concatenate(
        [h, jnp.zeros((nb, 96), jnp.float32)], axis=1)


def _node1(parts, x, root, bias, nb=1000):
    n = x.shape[0]
    n_pad = parts[0].shape[0] // 2
    ps = [p.reshape(2, n_pad, 128) for p in parts]
    return pl.pallas_call(
        functools.partial(_node1_body, nb, len(ps)),
        grid=(n // nb,),
        in_specs=(
            [pl.BlockSpec((2, nb, 128), lambda i: (0, i, 0))] * len(ps) + [
            pl.BlockSpec((nb, x.shape[1]), lambda i: (i, 0)),
            pl.BlockSpec(root.shape, lambda i: (0, 0)),
            pl.BlockSpec((1, 32), lambda i: (0, 0)),
        ]),
        out_specs=pl.BlockSpec((nb, 128), lambda i: (i, 0)),
        out_shape=jax.ShapeDtypeStruct((n, 128), jnp.float32),
    )(*ps, x, root, bias[None, :])


def _split_body(ei_ref, s_ref, d_ref):
    s_ref[...] = ei_ref[0]
    d_ref[...] = ei_ref[1]


def _split_edge_index(edge_index, blk=128000):
    e = edge_index.shape[1]
    out = jax.ShapeDtypeStruct((e,), jnp.int32)
    return pl.pallas_call(
        _split_body,
        grid=(e // blk,),
        in_specs=[pl.BlockSpec((2, blk), lambda i: (0, i))],
        out_specs=[pl.BlockSpec((blk,), lambda i: (i,)),
                   pl.BlockSpec((blk,), lambda i: (i,))],
        out_shape=[out, out],
    )(edge_index)


def _pool_body(nb, ns, *refs):
    p_refs = refs[:ns]
    h_ref, r_ref, b_ref, batch_ref, o_ref = refs[ns:]
    i = pl.program_id(0)
    agg = sum(p[0, :, 0:16] + p[1, :, 0:16] for p in p_refs)
    h2 = jnp.maximum(
        agg + jnp.dot(h_ref[:, 0:32], r_ref[...],
                      preferred_element_type=jnp.float32) + b_ref[...], 0.0)
    seg = lax.broadcasted_iota(jnp.int32, (_NUM_GRAPHS, nb), 0)
    onehot = jnp.where(seg == batch_ref[0], 1.0, 0.0)
    g = jnp.dot(onehot, h2, preferred_element_type=jnp.float32)

    @pl.when(i == 0)
    def _():
        o_ref[...] = jnp.zeros_like(o_ref)

    o_ref[...] += g


def _pool(parts2, h, root2, bias2, batch, nb=1000):
    n = h.shape[0]
    n_pad = parts2[0].shape[0] // 2
    ps = [p.reshape(2, n_pad, 128) for p in parts2]
    return pl.pallas_call(
        functools.partial(_pool_body, nb, len(ps)),
        grid=(n // nb,),
        in_specs=(
            [pl.BlockSpec((2, nb, 128), lambda i: (0, i, 0))] * len(ps) + [
            pl.BlockSpec((nb, 128), lambda i: (i, 0)),
            pl.BlockSpec(root2.shape, lambda i: (0, 0)),
            pl.BlockSpec((1, 16), lambda i: (0, 0)),
            pl.BlockSpec((1, 1, nb), lambda i: (i, 0, 0)),
        ]),
        out_specs=pl.BlockSpec((_NUM_GRAPHS, 16), lambda i: (0, 0)),
        out_shape=jax.ShapeDtypeStruct((_NUM_GRAPHS, 16), jnp.float32),
    )(*ps, h, root2, bias2[None, :], batch.reshape(n // nb, 1, nb))


def _mlp_body(g_ref, fw_ref, fb_ref, ow_ref, ob_ref, o_ref):
    g = jnp.maximum(
        jnp.dot(g_ref[...], fw_ref[...], preferred_element_type=jnp.float32)
        + fb_ref[...], 0.0)
    o_ref[...] = jnp.dot(g, ow_ref[...],
                         preferred_element_type=jnp.float32) + ob_ref[...]


def _mlp(g, fc1_w, fc1_b, out_w, out_b):
    return pl.pallas_call(
        _mlp_body,
        out_shape=jax.ShapeDtypeStruct((_NUM_GRAPHS, 1), jnp.float32),
    )(g, fc1_w, fc1_b[None, :], out_w, out_b[None, :])


def kernel(x, edge_index, edge_attr, batch, c1_W1, c1_b1, c1_W2, c1_b2,
           c1_root, c1_bias, c2_W1, c2_b1, c2_W2, c2_b2, c2_root, c2_bias,
           fc1_W, fc1_b, out_W, out_b):
    n = x.shape[0]
    e = edge_index.shape[1]
    eh = e // _NSTREAM
    blk = 2000
    n_pad = -(-n // (8 * _SC_SUBCORES)) * (8 * _SC_SUBCORES)
    src, dst = _split_edge_index(edge_index)

    x128 = jnp.pad(x, ((0, 0), (0, 128 - x.shape[1])))
    zeros = jnp.zeros((n_pad, 128), jnp.float32)

    p1 = []
    for s in range(_NSTREAM):
        xj = _sc_gather(x128, src, s * eh, eh)
        msg1 = _edge_messages(edge_attr, xj, c1_W1, c1_b1, c1_W2, c1_b2,
                              in_c=11, out_c=32, blk=blk,
                              blk_off=s * eh // blk)
        p1.append(_sc_scatter_add(msg1, dst, s * eh, zeros))
    h = _node1(p1, x, c1_root, c1_bias)

    p2 = []
    for s in range(_NSTREAM):
        hj = _sc_gather(h, src, s * eh, eh)
        msg2 = _edge_messages(edge_attr, hj, c2_W1, c2_b1, c2_W2, c2_b2,
                              in_c=32, out_c=16, blk=blk,
                              blk_off=s * eh // blk)
        p2.append(_sc_scatter_add(msg2, dst, s * eh, zeros))
    g = _pool(p2, h, c2_root, c2_bias, batch)
    return _mlp(g, fc1_W, fc1_b, out_W, out_b)


# speedup vs baseline: 1.2189x; 1.0068x over previous
import functools

import jax
import jax.numpy as jnp
from jax import lax
from jax.experimental import pallas as pl
from jax.experimental.pallas import tpu as pltpu
from jax.experimental.pallas import tpu_sc as plsc

_NUM_GRAPHS = 128
_SC_CORES = 2
_SC_SUBCORES = 16
_SC_WORKERS = _SC_CORES * _SC_SUBCORES
_NSTREAM = 4
_K = 40
_J = 5
_CHUNK = _K * _J
_SK = 40
_SJ = 5
_SCHUNK = _SK * _SJ


def _sc_gather(table, idx, e_off, e_len):
    e = e_len
    n_rows = table.shape[0]
    per_w = e // _SC_WORKERS
    stage = 640
    mesh = plsc.VectorSubcoreMesh(core_axis_name="c", subcore_axis_name="s")

    @functools.partial(
        pl.kernel,
        mesh=mesh,
        out_type=jax.ShapeDtypeStruct((e, 128), jnp.float32),
        scratch_types=[
            pltpu.VMEM((_CHUNK,), jnp.int32),
            pltpu.VMEM((_CHUNK, 128), jnp.float32),
            pltpu.VMEM_SHARED((n_rows, 128), jnp.float32),
            pltpu.SemaphoreType.DMA,
        ],
    )
    def k(table_hbm, idx_hbm, out_hbm, idx_v, rows_v, tab_sh, sem):
        cid = lax.axis_index("c")
        sid = lax.axis_index("s")
        wid = sid * _SC_CORES + cid
        base = wid * per_w

        full = n_rows // stage
        tail = n_rows - full * stage

        @pl.when(sid < full)
        def _():
            pltpu.sync_copy(table_hbm.at[pl.ds(sid * stage, stage)],
                            tab_sh.at[pl.ds(sid * stage, stage)])

        if tail:
            @pl.when(sid == full)
            def _():
                pltpu.sync_copy(table_hbm.at[pl.ds(full * stage, tail)],
                                tab_sh.at[pl.ds(full * stage, tail)])
        plsc.subcore_barrier()

        @pl.loop(0, per_w // _CHUNK)
        def _(t):
            pltpu.sync_copy(
                idx_hbm.at[pl.ds(e_off + base + t * _CHUNK, _CHUNK)],
                idx_v)
            handles = [
                pltpu.async_copy(tab_sh.at[idx_v.at[pl.ds(j * _K, _K)]],
                                 rows_v.at[pl.ds(j * _K, _K)], sem)
                for j in range(_J)
            ]
            for h in handles:
                h.wait()
            pltpu.sync_copy(rows_v,
                            out_hbm.at[pl.ds(base + t * _CHUNK, _CHUNK)])

    return k(table, idx)


def _sc_scatter_add(msg, dst, e_off, zeros):
    e, _ = msg.shape
    n_pad = zeros.shape[0]
    per_w = e // _SC_WORKERS
    rows_per_sub = n_pad // _SC_SUBCORES
    mesh = plsc.VectorSubcoreMesh(core_axis_name="c", subcore_axis_name="s")

    @functools.partial(
        pl.kernel,
        mesh=mesh,
        out_type=jax.ShapeDtypeStruct((2 * n_pad, 128), jnp.float32),
        scratch_types=[
            pltpu.VMEM((_SJ, _SK), jnp.int32),
            pltpu.VMEM((_SCHUNK, 128), jnp.float32),
            pltpu.VMEM_SHARED((n_pad, 128), jnp.float32),
            pltpu.SemaphoreType.DMA,
            pltpu.SemaphoreType.DMA,
        ],
    )
    def k(msg_hbm, dst_hbm, zero_hbm, out_hbm, idx_v, rows_v, acc_sh, sem,
          isem):
        cid = lax.axis_index("c")
        sid = lax.axis_index("s")
        wid = sid * _SC_CORES + cid
        base = wid * per_w
        my_rows = sid * rows_per_sub

        pltpu.sync_copy(zero_hbm.at[pl.ds(my_rows, rows_per_sub)],
                        acc_sh.at[pl.ds(my_rows, rows_per_sub)])
        plsc.subcore_barrier()

        @pl.loop(0, per_w // _SCHUNK)
        def _(t):
            ih = [
                pltpu.async_copy(
                    dst_hbm.at[
                        pl.ds(e_off + base + t * _SCHUNK + j * _SK, _SK)],
                    idx_v.at[j], isem)
                for j in range(_SJ)
            ]
            pltpu.sync_copy(msg_hbm.at[pl.ds(base + t * _SCHUNK, _SCHUNK)],
                            rows_v)
            for h in ih:
                h.wait()
            ah = [
                pltpu.async_copy(rows_v.at[pl.ds(j * _SK, _SK)],
                                 acc_sh.at[idx_v.at[j]], sem, add=True)
                for j in range(_SJ)
            ]
            for h in ah:
                h.wait()

        plsc.subcore_barrier()
        pltpu.sync_copy(
            acc_sh.at[pl.ds(my_rows, rows_per_sub)],
            out_hbm.at[pl.ds(cid * n_pad + my_rows, rows_per_sub)])

    return k(msg, dst, zeros)


def _edge_body(in_c, out_c, blk, ea_ref, xj_ref, w1_ref, b1_ref, w2_ref,
               b2_ref, rep_ref, sum_ref, out_ref):
    h = jnp.maximum(
        jnp.dot(ea_ref[...], w1_ref[...],
                preferred_element_type=jnp.float32) + b1_ref[...], 0.0)
    g = jnp.dot(h, w2_ref[...],
                preferred_element_type=jnp.float32) + b2_ref[...]
    xjr = jnp.dot(xj_ref[...], rep_ref[...],
                  preferred_element_type=jnp.float32)
    msg = jnp.dot(xjr * g, sum_ref[...], preferred_element_type=jnp.float32)
    out_ref[...] = jnp.concatenate(
        [msg, jnp.zeros((blk, 128 - out_c), jnp.float32)], axis=1)


def _edge_messages(ea, xj, w1, b1, w2, b2, in_c, out_c, blk, blk_off):
    e = xj.shape[0]
    hid = w1.shape[1]
    ic_oc = in_c * out_c
    rep = (jnp.arange(ic_oc)[None, :] // out_c
           == jnp.arange(128)[:, None]).astype(jnp.float32)
    summ = (jnp.arange(ic_oc)[:, None] % out_c
            == jnp.arange(out_c)[None, :]).astype(jnp.float32)
    kfn = functools.partial(_edge_body, in_c, out_c, blk)
    return pl.pallas_call(
        kfn,
        grid=(e // blk,),
        in_specs=[
            pl.BlockSpec((blk, ea.shape[1]), lambda i: (i + blk_off, 0)),
            pl.BlockSpec((blk, 128), lambda i: (i, 0)),
            pl.BlockSpec(w1.shape, lambda i: (0, 0)),
            pl.BlockSpec((1, hid), lambda i: (0, 0)),
            pl.BlockSpec(w2.shape, lambda i: (0, 0)),
            pl.BlockSpec((1, ic_oc), lambda i: (0, 0)),
            pl.BlockSpec((128, ic_oc), lambda i: (0, 0)),
            pl.BlockSpec((ic_oc, out_c), lambda i: (0, 0)),
        ],
        out_specs=pl.BlockSpec((blk, 128), lambda i: (i, 0)),
        out_shape=jax.ShapeDtypeStruct((e, 128), jnp.float32),
    )(ea, xj, w1, b1[None, :], w2, b2[None, :], rep, summ)


def _node1_body(nb, ns, *refs):
    p_refs = refs[:ns]
    x_ref, r_ref, b_ref, o_ref = refs[ns:]
    agg = sum(p[0, :, 0:32] + p[1, :, 0:32] for p in p_refs)
    h = jnp.maximum(
        agg + jnp.dot(x_ref[...], r_ref[...],
                      preferred_element_type=jnp.float32) + b_ref[...], 0.0)
    o_ref[...] = jnp.concatenate(
        [h, jnp.zeros((nb, 96), jnp.float32)], axis=1)


def _node1(parts, x, root, bias, nb=1000):
    n = x.shape[0]
    n_pad = parts[0].shape[0] // 2
    ps = [p.reshape(2, n_pad, 128) for p in parts]
    return pl.pallas_call(
        functools.partial(_node1_body, nb, len(ps)),
        grid=(n // nb,),
        in_specs=(
            [pl.BlockSpec((2, nb, 128), lambda i: (0, i, 0))] * len(ps) + [
            pl.BlockSpec((nb, x.shape[1]), lambda i: (i, 0)),
            pl.BlockSpec(root.shape, lambda i: (0, 0)),
            pl.BlockSpec((1, 32), lambda i: (0, 0)),
        ]),
        out_specs=pl.BlockSpec((nb, 128), lambda i: (i, 0)),
        out_shape=jax.ShapeDtypeStruct((n, 128), jnp.float32),
    )(*ps, x, root, bias[None, :])


def _split_body(ei_ref, s_ref, d_ref):
    s_ref[...] = ei_ref[0]
    d_ref[...] = ei_ref[1]


def _split_edge_index(edge_index, blk=128000):
    e = edge_index.shape[1]
    out = jax.ShapeDtypeStruct((e,), jnp.int32)
    return pl.pallas_call(
        _split_body,
        grid=(e // blk,),
        in_specs=[pl.BlockSpec((2, blk), lambda i: (0, i))],
        out_specs=[pl.BlockSpec((blk,), lambda i: (i,)),
                   pl.BlockSpec((blk,), lambda i: (i,))],
        out_shape=[out, out],
    )(edge_index)


def _pool_body(nb, ns, *refs):
    p_refs = refs[:ns]
    h_ref, r_ref, b_ref, batch_ref, o_ref = refs[ns:]
    i = pl.program_id(0)
    agg = sum(p[0, :, 0:16] + p[1, :, 0:16] for p in p_refs)
    h2 = jnp.maximum(
        agg + jnp.dot(h_ref[:, 0:32], r_ref[...],
                      preferred_element_type=jnp.float32) + b_ref[...], 0.0)
    seg = lax.broadcasted_iota(jnp.int32, (_NUM_GRAPHS, nb), 0)
    onehot = jnp.where(seg == batch_ref[0], 1.0, 0.0)
    g = jnp.dot(onehot, h2, preferred_element_type=jnp.float32)

    @pl.when(i == 0)
    def _():
        o_ref[...] = jnp.zeros_like(o_ref)

    o_ref[...] += g


def _pool(parts2, h, root2, bias2, batch, nb=1000):
    n = h.shape[0]
    n_pad = parts2[0].shape[0] // 2
    ps = [p.reshape(2, n_pad, 128) for p in parts2]
    return pl.pallas_call(
        functools.partial(_pool_body, nb, len(ps)),
        grid=(n // nb,),
        in_specs=(
            [pl.BlockSpec((2, nb, 128), lambda i: (0, i, 0))] * len(ps) + [
            pl.BlockSpec((nb, 128), lambda i: (i, 0)),
            pl.BlockSpec(root2.shape, lambda i: (0, 0)),
            pl.BlockSpec((1, 16), lambda i: (0, 0)),
            pl.BlockSpec((1, 1, nb), lambda i: (i, 0, 0)),
        ]),
        out_specs=pl.BlockSpec((_NUM_GRAPHS, 16), lambda i: (0, 0)),
        out_shape=jax.ShapeDtypeStruct((_NUM_GRAPHS, 16), jnp.float32),
    )(*ps, h, root2, bias2[None, :], batch.reshape(n // nb, 1, nb))


def _mlp_body(g_ref, fw_ref, fb_ref, ow_ref, ob_ref, o_ref):
    g = jnp.maximum(
        jnp.dot(g_ref[...], fw_ref[...], preferred_element_type=jnp.float32)
        + fb_ref[...], 0.0)
    o_ref[...] = jnp.dot(g, ow_ref[...],
                         preferred_element_type=jnp.float32) + ob_ref[...]


def _mlp(g, fc1_w, fc1_b, out_w, out_b):
    return pl.pallas_call(
        _mlp_body,
        out_shape=jax.ShapeDtypeStruct((_NUM_GRAPHS, 1), jnp.float32),
    )(g, fc1_w, fc1_b[None, :], out_w, out_b[None, :])


def kernel(x, edge_index, edge_attr, batch, c1_W1, c1_b1, c1_W2, c1_b2,
           c1_root, c1_bias, c2_W1, c2_b1, c2_W2, c2_b2, c2_root, c2_bias,
           fc1_W, fc1_b, out_W, out_b):
    n = x.shape[0]
    e = edge_index.shape[1]
    eh = e // _NSTREAM
    blk = 2000
    n_pad = -(-n // (8 * _SC_SUBCORES)) * (8 * _SC_SUBCORES)
    src, dst = _split_edge_index(edge_index)

    x128 = jnp.pad(x, ((0, 0), (0, 128 - x.shape[1])))
    zeros = jnp.zeros((n_pad, 128), jnp.float32)

    p1 = []
    for s in range(_NSTREAM):
        xj = _sc_gather(x128, src, s * eh, eh)
        msg1 = _edge_messages(edge_attr, xj, c1_W1, c1_b1, c1_W2, c1_b2,
                              in_c=11, out_c=32, blk=blk,
                              blk_off=s * eh // blk)
        p1.append(_sc_scatter_add(msg1, dst, s * eh, zeros))
    h = _node1(p1, x, c1_root, c1_bias)

    p2 = []
    for s in range(_NSTREAM):
        hj = _sc_gather(h, src, s * eh, eh)
        msg2 = _edge_messages(edge_attr, hj, c2_W1, c2_b1, c2_W2, c2_b2,
                              in_c=32, out_c=16, blk=blk,
                              blk_off=s * eh // blk)
        p2.append(_sc_scatter_add(msg2, dst, s * eh, zeros))
    g = _pool(p2, h, c2_root, c2_bias, batch)
    return _mlp(g, fc1_W, fc1_b, out_W, out_b)
